# baseline (speedup 1.0000x reference)
import functools

import jax
import jax.numpy as jnp
from jax.experimental import pallas as pl
from jax.experimental.pallas import tpu as pltpu

_G = 128


def _gap_kernel(x_ref, o_ref, *, hw, inv_hw):
    row_len = _G * hw
    j = jax.lax.broadcasted_iota(jnp.int32, (row_len, _G), 0)
    g = jax.lax.broadcasted_iota(jnp.int32, (row_len, _G), 1)
    sel = jnp.where(j // hw == g, inv_hw, 0.0).astype(jnp.float32)
    o_ref[...] = jnp.dot(
        x_ref[...].astype(jnp.float32), sel,
        preferred_element_type=jnp.float32,
    ).astype(o_ref.dtype)


def kernel(x: jax.Array) -> jax.Array:
    B, C, H, W = x.shape
    rows = B * C
    hw = H * W
    rows_p = pl.cdiv(rows, _G) * _G

    xf = x.reshape(-1)
    if rows_p != rows:
        xf = jnp.pad(xf, (0, (rows_p - rows) * hw))
    m = rows_p // _G
    row_len = _G * hw
    x2 = xf.reshape(m, row_len)

    tile_m = min(m, 256)
    while m % tile_m:
        tile_m //= 2
    grid = (m // tile_m,)

    itemsize = x2.dtype.itemsize
    cost = pl.CostEstimate(
        flops=2 * m * row_len * _G,
        transcendentals=0,
        bytes_accessed=m * row_len * itemsize + rows_p * itemsize,
    )

    out = pl.pallas_call(
        functools.partial(_gap_kernel, hw=hw, inv_hw=1.0 / float(hw)),
        out_shape=jax.ShapeDtypeStruct((m, _G), x.dtype),
        grid=grid,
        in_specs=[pl.BlockSpec((tile_m, row_len), lambda i: (i, 0))],
        out_specs=pl.BlockSpec((tile_m, _G), lambda i: (i, 0)),
        compiler_params=pltpu.CompilerParams(
            dimension_semantics=("parallel",),
        ),
        cost_estimate=cost,
    )(x2)

    return out.reshape(rows_p)[:rows].reshape(B, C, 1, 1)


# speedup vs baseline: 29.4789x; 29.4789x over previous
import functools

import jax
import jax.numpy as jnp
from jax.experimental import pallas as pl
from jax.experimental.pallas import tpu as pltpu


def _gap_kernel(x_ref, o_ref, *, inv_hw, nk):
    k = pl.program_id(1)

    @pl.when(k == 0)
    def _init():
        o_ref[...] = jnp.zeros_like(o_ref)

    o_ref[...] += jnp.sum(x_ref[...].astype(jnp.float32), axis=0)

    @pl.when(k == nk - 1)
    def _finalize():
        o_ref[...] *= inv_hw


def kernel(x: jax.Array) -> jax.Array:
    B, C, H, W = x.shape
    hw = H * W
    y = jnp.transpose(x, (2, 3, 0, 1)).reshape(hw, B, C)

    tile_b = B
    for cand in (64, 32, 16, 8):
        if B % cand == 0:
            tile_b = cand
            break
    grid = (B // tile_b, H)

    itemsize = x.dtype.itemsize
    cost = pl.CostEstimate(
        flops=hw * B * C,
        transcendentals=0,
        bytes_accessed=hw * B * C * itemsize + B * C * itemsize,
    )

    out = pl.pallas_call(
        functools.partial(_gap_kernel, inv_hw=1.0 / float(hw), nk=H),
        out_shape=jax.ShapeDtypeStruct((B, C), jnp.float32),
        grid=grid,
        in_specs=[pl.BlockSpec((W, tile_b, C), lambda i, k: (k, i, 0))],
        out_specs=pl.BlockSpec((tile_b, C), lambda i, k: (i, 0)),
        compiler_params=pltpu.CompilerParams(
            dimension_semantics=("parallel", "arbitrary"),
        ),
        cost_estimate=cost,
    )(y)

    return out.astype(x.dtype).reshape(B, C, 1, 1)


# speedup vs baseline: 32.1587x; 1.0909x over previous
import functools

import jax
import jax.numpy as jnp
from jax.experimental import pallas as pl
from jax.experimental.pallas import tpu as pltpu


def _gap_kernel(x_ref, o_ref, *, inv_hw, nk):
    k = pl.program_id(1)

    @pl.when(k == 0)
    def _init():
        o_ref[...] = jnp.zeros_like(o_ref)

    o_ref[...] += jnp.sum(x_ref[...].astype(jnp.float32), axis=0)

    @pl.when(k == nk - 1)
    def _finalize():
        o_ref[...] *= inv_hw


def kernel(x: jax.Array) -> jax.Array:
    B, C, H, W = x.shape
    hw = H * W
    y = jnp.transpose(x, (2, 3, 0, 1)).reshape(hw, B, C)

    tile_b = B
    for cand in (128, 64, 32, 16, 8):
        if B % cand == 0:
            tile_b = cand
            break
    grid = (B // tile_b, H)

    itemsize = x.dtype.itemsize
    cost = pl.CostEstimate(
        flops=hw * B * C,
        transcendentals=0,
        bytes_accessed=hw * B * C * itemsize + B * C * itemsize,
    )

    out = pl.pallas_call(
        functools.partial(_gap_kernel, inv_hw=1.0 / float(hw), nk=H),
        out_shape=jax.ShapeDtypeStruct((B, C), jnp.float32),
        grid=grid,
        in_specs=[pl.BlockSpec((W, tile_b, C), lambda i, k: (k, i, 0))],
        out_specs=pl.BlockSpec((tile_b, C), lambda i, k: (i, 0)),
        compiler_params=pltpu.CompilerParams(
            dimension_semantics=("parallel", "arbitrary"),
        ),
        cost_estimate=cost,
    )(y)

    return out.astype(x.dtype).reshape(B, C, 1, 1)
